# baseline (speedup 1.0000x reference)
import jax
import jax.numpy as jnp
from jax import lax
from jax.experimental import pallas as pl
from jax.experimental.pallas import tpu as pltpu
from jax.experimental.pallas import tpu_sc as plsc

_B = 16384
_D = 64
_F = 128
_NC = 2
_NS = 16
_NW = _NC * _NS
_NIDX = 2 * _B
_PER_W = _NIDX // _NW
_CHUNK = 128
_NCHUNK = _PER_W // _CHUNK

_mesh = plsc.VectorSubcoreMesh(
    core_axis_name="c", subcore_axis_name="s",
    num_cores=_NC, num_subcores=_NS,
)


def _gather_body(emb_hbm, idx_hbm, out_hbm, idx_v, rows_v, sem):
    wid = lax.axis_index("s") * _NC + lax.axis_index("c")
    pltpu.sync_copy(idx_hbm.at[wid], idx_v)
    cps = [
        pltpu.async_copy(
            emb_hbm.at[idx_v.at[j]],
            rows_v.at[pl.ds(j * _CHUNK, _CHUNK)],
            sem,
        )
        for j in range(_NCHUNK)
    ]
    for cp in cps:
        cp.wait()
    pltpu.sync_copy(rows_v, out_hbm.at[pl.ds(wid * _PER_W, _PER_W)])


_gather = pl.kernel(
    _gather_body,
    out_type=jax.ShapeDtypeStruct((_NIDX, _D), jnp.float32),
    mesh=_mesh,
    scratch_types=[
        pltpu.VMEM((_NCHUNK, _CHUNK), jnp.int32),
        pltpu.VMEM((_PER_W, _D), jnp.float32),
        pltpu.SemaphoreType.DMA,
    ],
    compiler_params=pltpu.CompilerParams(use_tc_tiling_on_sc=False),
)

_BT = 2048


def _mlp_body(g_ref, w1t_ref, b1_ref, w2t_ref, b2_ref, out_ref):
    h = jnp.dot(g_ref[...], w1t_ref[...], preferred_element_type=jnp.float32)
    h = jnp.tanh(h + b1_ref[...])
    out_ref[...] = (
        jnp.dot(h, w2t_ref[...], preferred_element_type=jnp.float32)
        + b2_ref[...]
    )


def _mlp(g, w1t, b1r, w2t, b2r):
    return pl.pallas_call(
        _mlp_body,
        grid=(_B // _BT,),
        in_specs=[
            pl.BlockSpec((_BT, 2 * _D), lambda i: (i, 0)),
            pl.BlockSpec((2 * _D, _F), lambda i: (0, 0)),
            pl.BlockSpec((1, _F), lambda i: (0, 0)),
            pl.BlockSpec((_F, 2), lambda i: (0, 0)),
            pl.BlockSpec((1, 2), lambda i: (0, 0)),
        ],
        out_specs=pl.BlockSpec((_BT, 2), lambda i: (i, 0)),
        out_shape=jax.ShapeDtypeStruct((_B, 2), jnp.float32),
    )(g, w1t, b1r, w2t, b2r)


def kernel(x, x_mask, ents, batch_spos, batch_tpos, batch_sent_chars, emb, W1, b1, W2, b2):
    idx = ents.astype(jnp.int32).reshape(_NW, _NCHUNK, _CHUNK)
    g = _gather(emb, idx).reshape(_B, 2 * _D)
    return _mlp(g, W1.T, b1.reshape(1, _F), W2.T, b2.reshape(1, 2))
